# baseline (speedup 1.0000x reference)
import functools
import math

import jax
import jax.numpy as jnp
from jax import lax
from jax.experimental import pallas as pl
from jax.experimental.pallas import tpu as pltpu
from jax.experimental.pallas import tpu_sc as plsc

DIM = 128
HALF = DIM // 2
PE_T = 50
PE_TP = 56

NC = 2
NS = 16
NW = NC * NS

GB = 8
C = GB * PE_TP


def _pe_body(out_ref):
    t = lax.broadcasted_iota(jnp.int32, (PE_T, DIM), 0).astype(jnp.float32)
    d = lax.broadcasted_iota(jnp.int32, (PE_T, DIM), 1)
    dh = jnp.where(d < HALF, d, d - HALF).astype(jnp.float32)
    freq = jnp.exp(dh * (-2.0 * math.log(10000.0) / DIM))
    angle = t * freq
    out_ref[...] = jnp.where(d < HALF, jnp.sin(angle), jnp.cos(angle))


def _make_sc_kernel(n_batch):
    per_w = n_batch // NW
    n_chunks = per_w // GB
    mesh = plsc.VectorSubcoreMesh(core_axis_name="c", subcore_axis_name="s")

    @functools.partial(
        pl.kernel,
        mesh=mesh,
        out_type=jax.ShapeDtypeStruct((n_batch, PE_T, DIM), jnp.float32),
        scratch_types=[
            pltpu.VMEM((C,), jnp.int32),
            pltpu.VMEM((C,), jnp.int32),
            pltpu.VMEM((GB, PE_T, DIM), jnp.float32),
            pltpu.VMEM((GB, PE_T, DIM), jnp.float32),
            pltpu.VMEM((PE_T, DIM), jnp.float32),
            pltpu.SemaphoreType.DMA,
            pltpu.SemaphoreType.DMA,
            pltpu.SemaphoreType.DMA,
            pltpu.SemaphoreType.DMA,
        ],
        compiler_params=pltpu.CompilerParams(use_tc_tiling_on_sc=True),
    )
    def body(ids_hbm, pe_hbm, matrix_hbm, out_hbm,
             idx0, idx1, buf0, buf1, pe_v, gsem0, gsem1, osem0, osem1):
        wid = lax.axis_index("s") * NC + lax.axis_index("c")
        base = wid * per_w
        pltpu.sync_copy(pe_hbm, pe_v)

        idxs = (idx0, idx1)
        bufs = (buf0, buf1)
        gsems = (gsem0, gsem1)
        osems = (osem0, osem1)

        def fire(ci, p):
            cbase = (base + ci * GB) * PE_TP
            pltpu.sync_copy(ids_hbm.at[pl.ds(cbase, C)], idxs[p])
            return [
                pltpu.async_copy(
                    matrix_hbm.at[idxs[p].at[pl.ds(k * PE_TP, PE_T)]],
                    bufs[p].at[k],
                    gsems[p],
                )
                for k in range(GB)
            ]

        def add_pe(p):
            buf = bufs[p]

            def t_body(t, carry):
                for j in range(DIM // 16):
                    sl = pl.ds(j * 16, 16)
                    pe_reg = pe_v[t, sl]
                    for k in range(GB):
                        buf[k, t, sl] += pe_reg
                return carry

            lax.fori_loop(0, PE_T, t_body, 0)

        gh = [None, None]
        oh = [None, None]
        gh[0] = fire(0, 0)
        for ci in range(n_chunks):
            p = ci % 2
            q = 1 - p
            if ci + 1 < n_chunks:
                if oh[q] is not None:
                    oh[q].wait()
                    oh[q] = None
                gh[q] = fire(ci + 1, q)
            for h in gh[p]:
                h.wait()
            add_pe(p)
            oh[p] = pltpu.async_copy(
                bufs[p], out_hbm.at[pl.ds(base + ci * GB, GB)], osems[p]
            )
        for h in oh:
            if h is not None:
                h.wait()

    return body


def kernel(ids, matrix):
    b, hist = ids.shape
    ids_fix = jnp.sign(ids + 1) * ids
    ids_pad = jnp.pad(ids_fix, ((0, 0), (0, PE_TP - hist))).reshape(-1)
    pe = pl.pallas_call(
        _pe_body,
        out_shape=jax.ShapeDtypeStruct((PE_T, DIM), jnp.float32),
    )()
    half = b // 2
    sc = _make_sc_kernel(half)
    o1 = sc(ids_pad[: half * PE_TP], pe, matrix)
    o2 = sc(ids_pad[half * PE_TP:], pe, matrix)
    return jnp.concatenate([o1, o2], axis=0)


# speedup vs baseline: 1.5985x; 1.5985x over previous
import functools
import math

import jax
import jax.numpy as jnp
from jax import lax
from jax.experimental import pallas as pl
from jax.experimental.pallas import tpu as pltpu
from jax.experimental.pallas import tpu_sc as plsc

DIM = 128
HALF = DIM // 2
PE_T = 50
PE_TP = 56

NC = 2
NS = 16
NW = NC * NS

GB = 8
C = GB * PE_TP


def _pe_body(out_ref):
    t = lax.broadcasted_iota(jnp.int32, (PE_T, DIM), 0).astype(jnp.float32)
    d = lax.broadcasted_iota(jnp.int32, (PE_T, DIM), 1)
    dh = jnp.where(d < HALF, d, d - HALF).astype(jnp.float32)
    freq = jnp.exp(dh * (-2.0 * math.log(10000.0) / DIM))
    angle = t * freq
    out_ref[...] = jnp.where(d < HALF, jnp.sin(angle), jnp.cos(angle))


def _make_sc_kernel(n_batch):
    per_w = n_batch // NW
    n_chunks = per_w // GB
    mesh = plsc.VectorSubcoreMesh(core_axis_name="c", subcore_axis_name="s")

    @functools.partial(
        pl.kernel,
        mesh=mesh,
        out_type=jax.ShapeDtypeStruct((n_batch, PE_T, DIM), jnp.float32),
        scratch_types=[
            pltpu.VMEM((C,), jnp.int32),
            pltpu.VMEM((C,), jnp.int32),
            pltpu.VMEM((GB, PE_T, DIM), jnp.float32),
            pltpu.VMEM((GB, PE_T, DIM), jnp.float32),
            pltpu.VMEM((PE_T, DIM), jnp.float32),
            pltpu.SemaphoreType.DMA,
            pltpu.SemaphoreType.DMA,
            pltpu.SemaphoreType.DMA,
            pltpu.SemaphoreType.DMA,
        ],
        compiler_params=pltpu.CompilerParams(use_tc_tiling_on_sc=True),
    )
    def body(ids_hbm, pe_hbm, matrix_hbm, out_hbm,
             idx0, idx1, buf0, buf1, pe_v, gsem0, gsem1, osem0, osem1):
        wid = lax.axis_index("s") * NC + lax.axis_index("c")
        base = wid * per_w
        pltpu.sync_copy(pe_hbm, pe_v)

        idxs = (idx0, idx1)
        bufs = (buf0, buf1)
        gsems = (gsem0, gsem1)
        osems = (osem0, osem1)

        def fire(ci, p):
            cbase = (base + ci * GB) * PE_TP
            pltpu.sync_copy(ids_hbm.at[pl.ds(cbase, C)], idxs[p])
            return [
                pltpu.async_copy(
                    matrix_hbm.at[idxs[p].at[pl.ds(k * PE_TP, PE_T)]],
                    bufs[p].at[k],
                    gsems[p],
                )
                for k in range(GB)
            ]

        def add_pe(p):
            buf = bufs[p]

            def t_body(t, carry):
                for j in range(DIM // 16):
                    sl = pl.ds(j * 16, 16)
                    pe_reg = pe_v[t, sl]
                    for k in range(GB):
                        buf[k, t, sl] += pe_reg
                return carry

            lax.fori_loop(0, PE_T, t_body, 0)

        gh = [None, None]
        oh = [None, None]
        gh[0] = fire(0, 0)
        for ci in range(n_chunks):
            p = ci % 2
            q = 1 - p
            if ci + 1 < n_chunks:
                if oh[q] is not None:
                    oh[q].wait()
                    oh[q] = None
                gh[q] = fire(ci + 1, q)
            for h in gh[p]:
                h.wait()
            add_pe(p)
            oh[p] = pltpu.async_copy(
                bufs[p], out_hbm.at[pl.ds(base + ci * GB, GB)], osems[p]
            )
        for h in oh:
            if h is not None:
                h.wait()

    return body


def kernel(ids, matrix):
    b, hist = ids.shape
    ids_fix = jnp.sign(ids + 1) * ids
    ids_pad = jnp.pad(ids_fix, ((0, 0), (0, PE_TP - hist))).reshape(-1)
    pe = pl.pallas_call(
        _pe_body,
        out_shape=jax.ShapeDtypeStruct((PE_T, DIM), jnp.float32),
    )()
    return _make_sc_kernel(b)(ids_pad, pe, matrix)


# speedup vs baseline: 1.6437x; 1.0283x over previous
import functools
import math

import jax
import jax.numpy as jnp
from jax import lax
from jax.experimental import pallas as pl
from jax.experimental.pallas import tpu as pltpu
from jax.experimental.pallas import tpu_sc as plsc

DIM = 128
HALF = DIM // 2
PE_T = 50
PE_TP = 56

NC = 2
NS = 16
NW = NC * NS

GB = 8
C = GB * PE_TP


def _pe_body(out_ref):
    t = lax.broadcasted_iota(jnp.int32, (PE_T, DIM), 0).astype(jnp.float32)
    d = lax.broadcasted_iota(jnp.int32, (PE_T, DIM), 1)
    dh = jnp.where(d < HALF, d, d - HALF).astype(jnp.float32)
    freq = jnp.exp(dh * (-2.0 * math.log(10000.0) / DIM))
    angle = t * freq
    out_ref[...] = jnp.where(d < HALF, jnp.sin(angle), jnp.cos(angle))


def _make_sc_kernel(n_batch):
    per_w = n_batch // NW
    n_chunks = per_w // GB
    mesh = plsc.VectorSubcoreMesh(core_axis_name="c", subcore_axis_name="s")

    @functools.partial(
        pl.kernel,
        mesh=mesh,
        out_type=jax.ShapeDtypeStruct((n_batch, PE_T, DIM), jnp.float32),
        scratch_types=[
            pltpu.VMEM((C,), jnp.int32),
            pltpu.VMEM((C,), jnp.int32),
            pltpu.VMEM((GB, PE_T, DIM), jnp.float32),
            pltpu.VMEM((GB, PE_T, DIM), jnp.float32),
            pltpu.VMEM((PE_T, DIM), jnp.float32),
            pltpu.SemaphoreType.DMA,
            pltpu.SemaphoreType.DMA,
            pltpu.SemaphoreType.DMA,
            pltpu.SemaphoreType.DMA,
        ],
        compiler_params=pltpu.CompilerParams(
            use_tc_tiling_on_sc=True, needs_layout_passes=True
        ),
    )
    def body(ids_hbm, pe_hbm, matrix_hbm, out_hbm,
             idx0, idx1, buf0, buf1, pe_v, gsem0, gsem1, osem0, osem1):
        wid = lax.axis_index("s") * NC + lax.axis_index("c")
        base = wid * per_w
        pltpu.sync_copy(pe_hbm, pe_v)

        idxs = (idx0, idx1)
        bufs = (buf0, buf1)
        gsems = (gsem0, gsem1)
        osems = (osem0, osem1)

        def fire(ci, p):
            cbase = (base + ci * GB) * PE_TP
            pltpu.sync_copy(ids_hbm.at[pl.ds(cbase, C)], idxs[p])
            return [
                pltpu.async_copy(
                    matrix_hbm.at[idxs[p].at[pl.ds(k * PE_TP, PE_T)]],
                    bufs[p].at[k],
                    gsems[p],
                )
                for k in range(GB)
            ]

        def add_pe(p):
            buf = bufs[p]

            def t_body(t, carry):
                for j in range(DIM // 16):
                    sl = pl.ds(j * 16, 16)
                    pe_reg = pe_v[t, sl]
                    for k in range(GB):
                        buf[k, t, sl] += pe_reg
                return carry

            lax.fori_loop(0, PE_T, t_body, 0)

        gh = [None, None]
        oh = [None, None]
        gh[0] = fire(0, 0)
        for ci in range(n_chunks):
            p = ci % 2
            q = 1 - p
            if ci + 1 < n_chunks:
                if oh[q] is not None:
                    oh[q].wait()
                    oh[q] = None
                gh[q] = fire(ci + 1, q)
            for h in gh[p]:
                h.wait()
            add_pe(p)
            oh[p] = pltpu.async_copy(
                bufs[p], out_hbm.at[pl.ds(base + ci * GB, GB)], osems[p]
            )
        for h in oh:
            if h is not None:
                h.wait()

    return body


def kernel(ids, matrix):
    b, hist = ids.shape
    ids_fix = jnp.sign(ids + 1) * ids
    ids_pad = jnp.pad(ids_fix, ((0, 0), (0, PE_TP - hist))).reshape(-1)
    pe = pl.pallas_call(
        _pe_body,
        out_shape=jax.ShapeDtypeStruct((PE_T, DIM), jnp.float32),
    )()
    return _make_sc_kernel(b)(ids_pad, pe, matrix)
